# Initial kernel scaffold; baseline (speedup 1.0000x reference)
import jax
import jax.numpy as jnp
from jax.experimental import pallas as pl


def kernel(inputs, state, adj_src, adj_dst, adj_val, weights, bias):
    raise NotImplementedError("write your pallas kernel here")


# speedup vs baseline: 1.6095x; 1.6095x over previous
import functools

import jax
import jax.numpy as jnp
from jax import lax
from jax.experimental import pallas as pl
from jax.experimental.pallas import tpu as pltpu
from jax.experimental.pallas import tpu_sc as plsc

N_NODES = 10000
N_EDGES = 160000
FEAT = 128
BATCH = 4

NW = 32
E_PAD = 163840
CH = 128
CHUNKS_PER_TILE = E_PAD // NW // CH
N_PAD = 10240


def _proj_body(xi_ref, xs_ref, wi_ref, ws_ref, o_ref):
    o_ref[...] = (
        jnp.dot(xi_ref[...], wi_ref[...], preferred_element_type=jnp.float32)
        + jnp.dot(xs_ref[...], ws_ref[...], preferred_element_type=jnp.float32)
    )


def _project(xi, xs, wi, ws):
    return pl.pallas_call(
        _proj_body,
        grid=(BATCH * N_NODES // 1000,),
        in_specs=[
            pl.BlockSpec((1000, 128), lambda i: (i, 0)),
            pl.BlockSpec((1000, 128), lambda i: (i, 0)),
            pl.BlockSpec((128, 128), lambda i: (0, 0)),
            pl.BlockSpec((128, 128), lambda i: (0, 0)),
        ],
        out_specs=pl.BlockSpec((1000, 128), lambda i: (i, 0)),
        out_shape=jax.ShapeDtypeStruct((BATCH * N_NODES, 128), jnp.float32),
    )(xi, xs, wi, ws)


def _comb_body(p0_ref, p1_ref, b_ref, o_ref):
    o_ref[...] = p0_ref[0] + p1_ref[0] + b_ref[...]


def _combine(partial, bias2d):
    return pl.pallas_call(
        _comb_body,
        grid=(BATCH, N_NODES // 1000),
        in_specs=[
            pl.BlockSpec((1, 1, 1000, 128), lambda b, j: (0, b, j, 0)),
            pl.BlockSpec((1, 1, 1000, 128), lambda b, j: (1, b, j, 0)),
            pl.BlockSpec((1, 128), lambda b, j: (0, 0)),
        ],
        out_specs=pl.BlockSpec((1, 1000, 128), lambda b, j: (b, j, 0)),
        out_shape=jax.ShapeDtypeStruct((BATCH, N_NODES, 128), jnp.float32),
    )(partial, partial, bias2d)


def _sc_spmm(y0, y1, y2, y3, srcm, dstm, valm):
    mesh = plsc.VectorSubcoreMesh(core_axis_name="c", subcore_axis_name="s")

    @functools.partial(
        pl.kernel,
        mesh=mesh,
        out_type=jax.ShapeDtypeStruct((2, BATCH, N_PAD, 128), jnp.float32),
        scratch_types=[
            pltpu.VMEM((CHUNKS_PER_TILE, CH), jnp.int32),
            pltpu.VMEM((CHUNKS_PER_TILE, CH), jnp.int32),
            pltpu.VMEM((CH, 16), jnp.float32),
            pltpu.VMEM((CH, 128), jnp.float32),
            pltpu.VMEM_SHARED((N_PAD, 128), jnp.float32),
            pltpu.SemaphoreType.DMA,
        ],
    )
    def spmm(y0h, y1h, y2h, y3h, srch, dsth, valh, out_h,
             src_v, dst_v, val_v, rows_v, acc, sem):
        c = lax.axis_index("c")
        s = lax.axis_index("s")
        gid = c * 16 + s
        erow = gid * CHUNKS_PER_TILE

        pltpu.sync_copy(srch.at[pl.ds(erow, CHUNKS_PER_TILE)], src_v)
        pltpu.sync_copy(dsth.at[pl.ds(erow, CHUNKS_PER_TILE)], dst_v)

        stripe = s * (N_PAD // 16)

        for b, ytab in enumerate((y0h, y1h, y2h, y3h)):
            @pl.loop(0, CH)
            def _zrow(e):
                for k in range(8):
                    rows_v[e, pl.ds(16 * k, 16)] = jnp.zeros((16,), jnp.float32)

            for r in range(N_PAD // 16 // CH):
                pltpu.sync_copy(rows_v, acc.at[pl.ds(stripe + r * CH, CH)])
            plsc.subcore_barrier()

            @pl.loop(0, CHUNKS_PER_TILE)
            def _chunk(j):
                pltpu.sync_copy(valh.at[erow + j], val_v)
                pltpu.async_copy(ytab.at[src_v.at[j]], rows_v, sem).wait()

                @pl.loop(0, CH)
                def _edge(e):
                    v = val_v[e, :]
                    for k in range(8):
                        sl = pl.ds(16 * k, 16)
                        rows_v[e, sl] = rows_v[e, sl] * v

                pltpu.sync_copy(rows_v, acc.at[dst_v.at[j]], add=True)

            plsc.subcore_barrier()

            pltpu.sync_copy(
                acc.at[pl.ds(stripe, N_PAD // 16)],
                out_h.at[c, b, pl.ds(stripe, N_PAD // 16)],
            )
            plsc.subcore_barrier()

    return spmm(y0, y1, y2, y3, srcm, dstm, valm)


def kernel(inputs, state, adj_src, adj_dst, adj_val, weights, bias):
    xi = inputs.reshape(BATCH * N_NODES, 128)
    xs = state.reshape(BATCH * N_NODES, 128)
    wi = weights[:128]
    ws = weights[128:]

    y = _project(xi, xs, wi, ws)
    y4 = y.reshape(BATCH, N_NODES, 128)

    pad = E_PAD - N_EDGES
    srcm = jnp.concatenate([adj_src, jnp.zeros((pad,), jnp.int32)]).reshape(-1, CH)
    dstm = jnp.concatenate([adj_dst, jnp.zeros((pad,), jnp.int32)]).reshape(-1, CH)
    valp = jnp.concatenate([adj_val, jnp.zeros((pad,), jnp.float32)])
    valm = jnp.broadcast_to(valp[:, None], (E_PAD, 16)).reshape(-1, CH, 16)

    partial = _sc_spmm(y4[0], y4[1], y4[2], y4[3], srcm, dstm, valm)

    out = _combine(partial, bias.reshape(1, 128))
    return out.reshape(BATCH, N_NODES * FEAT)


# speedup vs baseline: 2.0543x; 1.2764x over previous
import functools

import jax
import jax.numpy as jnp
from jax import lax
from jax.experimental import pallas as pl
from jax.experimental.pallas import tpu as pltpu
from jax.experimental.pallas import tpu_sc as plsc

N_NODES = 10000
N_EDGES = 160000
FEAT = 128
BATCH = 4

NW = 32
E_PAD = 163840
CH = 128
CHUNKS_PER_TILE = E_PAD // NW // CH
N_PAD = 10240


def _proj_body(xi_ref, xs_ref, wi_ref, ws_ref, o_ref):
    o_ref[...] = (
        jnp.dot(xi_ref[...], wi_ref[...], preferred_element_type=jnp.float32)
        + jnp.dot(xs_ref[...], ws_ref[...], preferred_element_type=jnp.float32)
    )


def _project(xi, xs, wi, ws):
    return pl.pallas_call(
        _proj_body,
        grid=(BATCH * N_NODES // 1000,),
        in_specs=[
            pl.BlockSpec((1000, 128), lambda i: (i, 0)),
            pl.BlockSpec((1000, 128), lambda i: (i, 0)),
            pl.BlockSpec((128, 128), lambda i: (0, 0)),
            pl.BlockSpec((128, 128), lambda i: (0, 0)),
        ],
        out_specs=pl.BlockSpec((1000, 128), lambda i: (i, 0)),
        out_shape=jax.ShapeDtypeStruct((BATCH * N_NODES, 128), jnp.float32),
    )(xi, xs, wi, ws)


def _comb_body(p0_ref, p1_ref, b_ref, o_ref):
    o_ref[...] = p0_ref[0] + p1_ref[0] + b_ref[...]


def _combine(partial, bias2d):
    return pl.pallas_call(
        _comb_body,
        grid=(BATCH, N_NODES // 1000),
        in_specs=[
            pl.BlockSpec((1, 1, 1000, 128), lambda b, j: (0, b, j, 0)),
            pl.BlockSpec((1, 1, 1000, 128), lambda b, j: (1, b, j, 0)),
            pl.BlockSpec((1, 128), lambda b, j: (0, 0)),
        ],
        out_specs=pl.BlockSpec((1, 1000, 128), lambda b, j: (b, j, 0)),
        out_shape=jax.ShapeDtypeStruct((BATCH, N_NODES, 128), jnp.float32),
    )(partial, partial, bias2d)


def _sc_spmm(y0, y1, y2, y3, srcm, dstm, valm):
    mesh = plsc.VectorSubcoreMesh(core_axis_name="c", subcore_axis_name="s")

    @functools.partial(
        pl.kernel,
        mesh=mesh,
        out_type=jax.ShapeDtypeStruct((2, BATCH, N_PAD, 128), jnp.float32),
        scratch_types=[
            pltpu.VMEM((2, CHUNKS_PER_TILE, CH), jnp.int32),
            pltpu.VMEM((CHUNKS_PER_TILE, CH), jnp.float32),
            pltpu.VMEM((2, CH, 128), jnp.float32),
            pltpu.VMEM_SHARED((N_PAD, 128), jnp.float32),
            pltpu.SemaphoreType.DMA,
            pltpu.SemaphoreType.DMA,
            pltpu.SemaphoreType.DMA,
            pltpu.SemaphoreType.DMA,
        ],
    )
    def spmm(y0h, y1h, y2h, y3h, srch, dsth, valh, out_h,
             idx_v, val_v, rows_v, acc,
             gsem0, gsem1, ssem0, ssem1):
        c = lax.axis_index("c")
        s = lax.axis_index("s")
        gid = c * 16 + s
        erow = gid * CHUNKS_PER_TILE

        pltpu.sync_copy(srch.at[pl.ds(erow, CHUNKS_PER_TILE)], idx_v.at[0])
        pltpu.sync_copy(dsth.at[pl.ds(erow, CHUNKS_PER_TILE)], idx_v.at[1])
        pltpu.sync_copy(valh.at[pl.ds(erow, CHUNKS_PER_TILE)], val_v)

        stripe = s * (N_PAD // 16)
        gsem = (gsem0, gsem1)
        ssem = (ssem0, ssem1)

        def scale_rows(p, j):
            @pl.loop(0, CH // 16)
            def _grp(g):
                v16 = val_v[j, pl.ds(16 * g, 16)]
                for l in range(16):
                    v = v16[l]
                    e = 16 * g + l
                    for k in range(8):
                        sl = pl.ds(16 * k, 16)
                        rows_v[p, e, sl] = rows_v[p, e, sl] * v

        def issue(ytab, p, j):
            pltpu.async_copy(ytab.at[idx_v.at[0, j]], rows_v.at[p], gsem[p])

        def wait_in(ytab, p, j):
            pltpu.make_async_copy(ytab.at[idx_v.at[0, j]], rows_v.at[p], gsem[p]).wait()

        def scatter(p, j):
            pltpu.async_copy(rows_v.at[p], acc.at[idx_v.at[1, j]], ssem[p], add=True)

        def wait_scatter(p, j):
            pltpu.make_async_copy(rows_v.at[p], acc.at[idx_v.at[1, j]], ssem[p]).wait()

        for b, ytab in enumerate((y0h, y1h, y2h, y3h)):
            @pl.loop(0, CH)
            def _zrow(e):
                for k in range(8):
                    rows_v[0, e, pl.ds(16 * k, 16)] = jnp.zeros((16,), jnp.float32)

            for r in range(N_PAD // 16 // CH):
                pltpu.sync_copy(rows_v.at[0], acc.at[pl.ds(stripe + r * CH, CH)])
            plsc.subcore_barrier()

            issue(ytab, 0, 0)
            issue(ytab, 1, 1)

            @pl.loop(0, CHUNKS_PER_TILE // 2)
            def _pair(t):
                j0 = 2 * t
                j1 = 2 * t + 1
                wait_in(ytab, 0, j0)
                scale_rows(0, j0)
                scatter(0, j0)

                wait_in(ytab, 1, j1)
                scale_rows(1, j1)
                scatter(1, j1)

                @pl.when(t < CHUNKS_PER_TILE // 2 - 1)
                def _refill():
                    wait_scatter(0, j0)
                    issue(ytab, 0, j0 + 2)
                    wait_scatter(1, j1)
                    issue(ytab, 1, j1 + 2)

            wait_scatter(0, 0)
            wait_scatter(1, 1)
            plsc.subcore_barrier()

            pltpu.sync_copy(
                acc.at[pl.ds(stripe, N_PAD // 16)],
                out_h.at[c, b, pl.ds(stripe, N_PAD // 16)],
            )
            plsc.subcore_barrier()

    return spmm(y0, y1, y2, y3, srcm, dstm, valm)


def kernel(inputs, state, adj_src, adj_dst, adj_val, weights, bias):
    xi = inputs.reshape(BATCH * N_NODES, 128)
    xs = state.reshape(BATCH * N_NODES, 128)
    wi = weights[:128]
    ws = weights[128:]

    y = _project(xi, xs, wi, ws)
    y4 = y.reshape(BATCH, N_NODES, 128)

    pad = E_PAD - N_EDGES
    srcm = jnp.concatenate([adj_src, jnp.zeros((pad,), jnp.int32)]).reshape(-1, CH)
    dstm = jnp.concatenate([adj_dst, jnp.zeros((pad,), jnp.int32)]).reshape(-1, CH)
    valm = jnp.concatenate([adj_val, jnp.zeros((pad,), jnp.float32)]).reshape(-1, CH)

    partial = _sc_spmm(y4[0], y4[1], y4[2], y4[3], srcm, dstm, valm)

    out = _combine(partial, bias.reshape(1, 128))
    return out.reshape(BATCH, N_NODES * FEAT)
